# baseline (speedup 1.0000x reference)
import functools

import jax
import jax.numpy as jnp
from jax import lax
from jax.experimental import pallas as pl
from jax.experimental.pallas import tpu as pltpu
from jax.experimental.pallas import tpu_sc as plsc

N = 10000
D = 128
E_RAW = 160000
ET = 170000
NC = 2
NS = 16
NW = NC * NS
CH = 224
NCHUNK = 24
EPW = CH * NCHUNK
E_PAD = EPW * NW
N_ACC = 10112
SPT = N_ACC // NS
NEG_SLOPE = 0.2
EPS = 1e-5
NEG_BIG = -1e30


def _mesh():
    return plsc.VectorSubcoreMesh(core_axis_name="c", subcore_axis_name="s",
                                  num_cores=NC, num_subcores=NS)


def _iota16():
    return lax.iota(jnp.int32, 16)


def _scatter_combine(ref, idx, val, op):
    def cond(act):
        return jnp.any(act)

    def body(act):
        _, last = plsc.scan_count(idx, mask=act)
        old = plsc.load_gather(ref, [idx], mask=last)
        plsc.store_scatter(ref, [idx], op(old, val), mask=last)
        return act & ~last

    lax.while_loop(cond, body, jnp.ones((16,), jnp.bool_))


def _pass_a_body(src_hbm, dst_hbm, xl_hbm, xr_hbm, att_hbm,
                 logits_hbm, mxp_hbm,
                 src0, src1, dst0, dst1, xl0, xl1, xr0, xr1, lg0, lg1,
                 att_v, mx_v,
                 sem_i0, sem_i1, sem_r0, sem_r1, sem_o0, sem_o1):
    cid = lax.axis_index("c")
    sid = lax.axis_index("s")
    wid = sid * NC + cid
    base0 = wid * EPW
    srcs, dsts = (src0, src1), (dst0, dst1)
    xls, xrs, lgs = (xl0, xl1), (xr0, xr1), (lg0, lg1)
    sem_is, sem_rs, sem_os = (sem_i0, sem_i1), (sem_r0, sem_r1), (sem_o0, sem_o1)
    pltpu.sync_copy(att_hbm, att_v)

    def init_body(j, c):
        mx_v[pl.ds(j * 16, 16)] = jnp.full((16,), NEG_BIG, jnp.float32)
        return c

    lax.fori_loop(0, N // 16, init_body, 0, unroll=8)
    iota = _iota16()

    def idx_slice(ref, k):
        return ref.at[pl.ds(base0 + k * CH, CH)]

    pltpu.sync_copy(idx_slice(src_hbm, 0), src0)
    pltpu.sync_copy(idx_slice(dst_hbm, 0), dst0)
    pltpu.async_copy(idx_slice(src_hbm, 1), src1, sem_i1)
    pltpu.async_copy(idx_slice(dst_hbm, 1), dst1, sem_i1)
    pltpu.async_copy(xl_hbm.at[src0], xl0, sem_r0)
    pltpu.async_copy(xr_hbm.at[dst0], xr0, sem_r0)

    def pair_body(j, carry):
        for ph in (0, 1):
            k = 2 * j + ph
            sp, dp = srcs[ph], dsts[ph]
            xlp, xrp, lgp = xls[ph], xrs[ph], lgs[ph]
            so, do_ = srcs[1 - ph], dsts[1 - ph]
            pltpu.make_async_copy(xl_hbm.at[sp], xlp, sem_rs[ph]).wait()
            pltpu.make_async_copy(xr_hbm.at[dp], xrp, sem_rs[ph]).wait()
            d16s = [dp[pl.ds(v * 16, 16)] for v in range(CH // 16)]
            @pl.when(j < NCHUNK // 2 - 1)
            def _():
                pltpu.async_copy(idx_slice(src_hbm, k + 2), sp, sem_is[ph])
                pltpu.async_copy(idx_slice(dst_hbm, k + 2), dp, sem_is[ph])
            def issue_rows():
                pltpu.make_async_copy(idx_slice(src_hbm, k + 1), so,
                                      sem_is[1 - ph]).wait()
                pltpu.make_async_copy(idx_slice(dst_hbm, k + 1), do_,
                                      sem_is[1 - ph]).wait()
                H = CH // 2
                for hh in (0, 1):
                    pltpu.async_copy(xl_hbm.at[so.at[pl.ds(hh * H, H)]],
                                     xls[1 - ph].at[pl.ds(hh * H, H)],
                                     sem_rs[1 - ph])
                    pltpu.async_copy(xr_hbm.at[do_.at[pl.ds(hh * H, H)]],
                                     xrs[1 - ph].at[pl.ds(hh * H, H)],
                                     sem_rs[1 - ph])
            if ph == 0:
                issue_rows()
            else:
                pl.when(j < NCHUNK // 2 - 1)(issue_rows)

            def col_body(c, accs):
                cc = (iota + c) & (D - 1)
                attc = plsc.load_gather(att_v, [cc])
                out = []
                for v in range(CH // 16):
                    rows = iota + v * 16
                    m = (plsc.load_gather(xlp, [rows, cc])
                         + plsc.load_gather(xrp, [rows, cc]))
                    m = jnp.where(m > 0, m, NEG_SLOPE * m)
                    out.append(accs[v] + m * attc)
                return tuple(out)

            accs = lax.fori_loop(
                0, D, col_body, tuple(jnp.zeros((16,), jnp.float32)
                                      for _ in range(CH // 16)), unroll=4)
            @pl.when(j > 0)
            def _():
                pltpu.make_async_copy(lgp, idx_slice(logits_hbm, k - 2),
                                      sem_os[ph]).wait()
            base = base0 + k * CH
            for v in range(CH // 16):
                ids = base + v * 16 + iota
                logit = jnp.where(ids < ET, accs[v], NEG_BIG)
                lgp[pl.ds(v * 16, 16)] = logit
                _scatter_combine(mx_v, d16s[v], logit, jnp.maximum)
            pltpu.async_copy(lgp, idx_slice(logits_hbm, k), sem_os[ph])
        return carry

    lax.fori_loop(0, NCHUNK // 2, pair_body, 0)
    pltpu.make_async_copy(lg0, idx_slice(logits_hbm, NCHUNK - 2),
                          sem_o0).wait()
    pltpu.make_async_copy(lg1, idx_slice(logits_hbm, NCHUNK - 1),
                          sem_o1).wait()
    pltpu.sync_copy(mx_v, mxp_hbm.at[wid])


def _pass_a(src, dst, xl, xr, att):
    f = functools.partial(
        pl.kernel,
        out_type=(jax.ShapeDtypeStruct((E_PAD,), jnp.float32),
                  jax.ShapeDtypeStruct((NW, N), jnp.float32)),
        mesh=_mesh(),
        compiler_params=pltpu.CompilerParams(needs_layout_passes=False),
        scratch_types=[
            pltpu.VMEM((CH,), jnp.int32),
            pltpu.VMEM((CH,), jnp.int32),
            pltpu.VMEM((CH,), jnp.int32),
            pltpu.VMEM((CH,), jnp.int32),
            pltpu.VMEM((CH, D), jnp.float32),
            pltpu.VMEM((CH, D), jnp.float32),
            pltpu.VMEM((CH, D), jnp.float32),
            pltpu.VMEM((CH, D), jnp.float32),
            pltpu.VMEM((CH,), jnp.float32),
            pltpu.VMEM((CH,), jnp.float32),
            pltpu.VMEM((D,), jnp.float32),
            pltpu.VMEM((N,), jnp.float32),
            pltpu.SemaphoreType.DMA,
            pltpu.SemaphoreType.DMA,
            pltpu.SemaphoreType.DMA,
            pltpu.SemaphoreType.DMA,
            pltpu.SemaphoreType.DMA,
            pltpu.SemaphoreType.DMA,
        ],
    )(_pass_a_body)
    return f(src, dst, xl, xr, att)


CHB = 112
NCHB = EPW // CHB


def _pass_b(src, dst, xl, logits, mx, zeros):
    def body(src_hbm, dst_hbm, xl_hbm, logits_hbm, mx_hbm, zeros_hbm,
             accp_hbm, denp_hbm,
             src0, src1, dst0, dst1, xl0, xl1, lgv0, lgv1, mx_v, den_v,
             acc_sh, sem_i0, sem_i1, sem_r0, sem_r1):
        cid = lax.axis_index("c")
        sid = lax.axis_index("s")
        wid = sid * NC + cid
        base0 = wid * EPW
        iota = _iota16()
        srcs, dsts = (src0, src1), (dst0, dst1)
        xlb, lgvs = (xl0, xl1), (lgv0, lgv1)
        sem_is, sem_rs = (sem_i0, sem_i1), (sem_r0, sem_r1)

        def init_body(j, c):
            den_v[pl.ds(j * 16, 16)] = jnp.zeros((16,), jnp.float32)
            return c

        lax.fori_loop(0, N // 16, init_body, 0, unroll=8)

        def idx_slice(ref, k):
            return ref.at[pl.ds(base0 + k * CHB, CHB)]

        pltpu.sync_copy(idx_slice(src_hbm, 0), src0)
        pltpu.sync_copy(idx_slice(dst_hbm, 0), dst0)
        pltpu.sync_copy(idx_slice(logits_hbm, 0), lgv0)
        pltpu.async_copy(idx_slice(src_hbm, 1), src1, sem_i1)
        pltpu.async_copy(idx_slice(dst_hbm, 1), dst1, sem_i1)
        pltpu.async_copy(idx_slice(logits_hbm, 1), lgv1, sem_i1)
        pltpu.async_copy(xl_hbm.at[src0], xl0, sem_r0)
        pltpu.sync_copy(zeros_hbm.at[pl.ds(sid * SPT, SPT)],
                        acc_sh.at[pl.ds(sid * SPT, SPT)])
        pltpu.sync_copy(mx_hbm, mx_v)
        plsc.subcore_barrier()

        def pair_body(j, carry):
            for ph in (0, 1):
                k = 2 * j + ph
                sp, dp, xlp, lgp = srcs[ph], dsts[ph], xlb[ph], lgvs[ph]
                pltpu.make_async_copy(xl_hbm.at[sp], xlp, sem_rs[ph]).wait()

                def issue_rows():
                    pltpu.make_async_copy(idx_slice(src_hbm, k + 1),
                                          srcs[1 - ph], sem_is[1 - ph]).wait()
                    pltpu.make_async_copy(idx_slice(dst_hbm, k + 1),
                                          dsts[1 - ph], sem_is[1 - ph]).wait()
                    pltpu.make_async_copy(idx_slice(logits_hbm, k + 1),
                                          lgvs[1 - ph], sem_is[1 - ph]).wait()
                    HB = CHB // 2
                    for hh in (0, 1):
                        pltpu.async_copy(
                            xl_hbm.at[srcs[1 - ph].at[pl.ds(hh * HB, HB)]],
                            xlb[1 - ph].at[pl.ds(hh * HB, HB)],
                            sem_rs[1 - ph])
                if ph == 0:
                    issue_rows()
                else:
                    pl.when(j < NCHB // 2 - 1)(issue_rows)

                exs = []
                for v in range(CHB // 16):
                    d16 = dp[pl.ds(v * 16, 16)]
                    mxg = plsc.load_gather(mx_v, [d16])
                    lg = lgp[pl.ds(v * 16, 16)]
                    ex = jnp.exp(lg - mxg)
                    exs.append(ex)
                    _scatter_combine(den_v, d16, ex, jnp.add)

                def col_body(c, carry2):
                    cc = (iota + c) & (D - 1)
                    for v in range(CHB // 16):
                        rows = iota + v * 16
                        xlv = plsc.load_gather(xlp, [rows, cc])
                        plsc.store_scatter(xlp, [rows, cc], xlv * exs[v])
                    return carry2

                lax.fori_loop(0, D, col_body, 0, unroll=4)
                pltpu.sync_copy(xlp, acc_sh.at[dp], add=True)
                @pl.when(j < NCHB // 2 - 1)
                def _():
                    pltpu.async_copy(idx_slice(src_hbm, k + 2), sp, sem_is[ph])
                    pltpu.async_copy(idx_slice(dst_hbm, k + 2), dp, sem_is[ph])
                    pltpu.async_copy(idx_slice(logits_hbm, k + 2), lgp,
                                     sem_is[ph])
            return carry

        lax.fori_loop(0, NCHB // 2, pair_body, 0)
        pltpu.sync_copy(den_v, denp_hbm.at[wid])
        plsc.subcore_barrier()
        pltpu.sync_copy(acc_sh.at[pl.ds(sid * SPT, SPT)],
                        accp_hbm.at[cid, pl.ds(sid * SPT, SPT)])

    f = functools.partial(
        pl.kernel,
        out_type=(jax.ShapeDtypeStruct((NC, N_ACC, D), jnp.float32),
                  jax.ShapeDtypeStruct((NW, N), jnp.float32)),
        mesh=_mesh(),
        compiler_params=pltpu.CompilerParams(needs_layout_passes=False),
        scratch_types=[
            pltpu.VMEM((CHB,), jnp.int32),
            pltpu.VMEM((CHB,), jnp.int32),
            pltpu.VMEM((CHB,), jnp.int32),
            pltpu.VMEM((CHB,), jnp.int32),
            pltpu.VMEM((CHB, D), jnp.float32),
            pltpu.VMEM((CHB, D), jnp.float32),
            pltpu.VMEM((CHB,), jnp.float32),
            pltpu.VMEM((CHB,), jnp.float32),
            pltpu.VMEM((N,), jnp.float32),
            pltpu.VMEM((N,), jnp.float32),
            pltpu.VMEM_SHARED((N_ACC, D), jnp.float32),
            pltpu.SemaphoreType.DMA,
            pltpu.SemaphoreType.DMA,
            pltpu.SemaphoreType.DMA,
            pltpu.SemaphoreType.DMA,
        ],
    )(body)
    return f(src, dst, xl, logits, mx, zeros)


def _mm_t(a, w):
    return lax.dot_general(a, w, (((1,), (1,)), ((), ())),
                           preferred_element_type=jnp.float32)


def _tc_init_body(x_ref, pw_ref, pb_ref, wl_ref, bl_ref, wr_ref, br_ref,
                  h_ref, xl_ref, xr_ref):
    h = _mm_t(x_ref[...], pw_ref[...]) + pb_ref[...]
    h_ref[...] = h
    xl_ref[...] = _mm_t(h, wl_ref[...]) + bl_ref[...]
    xr_ref[...] = _mm_t(h, wr_ref[...]) + br_ref[...]


def _tc_init(x, pw, pb, wl, bl, wr, br):
    return pl.pallas_call(
        _tc_init_body,
        out_shape=(jax.ShapeDtypeStruct((N, D), jnp.float32),) * 3,
    )(x, pw, pb[None, :], wl, bl[None, :], wr, br[None, :])


def _tc_max_body(mxp_ref, mx_ref):
    mx_ref[...] = jnp.max(mxp_ref[...], axis=0)


def _tc_max(mxp):
    return pl.pallas_call(
        _tc_max_body,
        out_shape=jax.ShapeDtypeStruct((N,), jnp.float32),
    )(mxp)


def _gat_out(acc_ref, denp_ref, bias_ref):
    num = acc_ref[0, :N] + acc_ref[1, :N]
    den = jnp.sum(denp_ref[...], axis=0)[:, None]
    return num / (den + 1e-16) + bias_ref[...]


def _tc_mid_body(acc_ref, denp_ref, bias_ref, g_ref, b_ref, wl_ref, bl_ref,
                 wr_ref, br_ref, xl_ref, xr_ref):
    xg = _gat_out(acc_ref, denp_ref, bias_ref)
    mean = jnp.mean(xg)
    var = jnp.mean((xg - mean) ** 2)
    xg = (xg - mean) / jnp.sqrt(var + EPS) * g_ref[...] + b_ref[...]
    xg = jnp.maximum(xg, 0.0)
    xl_ref[...] = _mm_t(xg, wl_ref[...]) + bl_ref[...]
    xr_ref[...] = _mm_t(xg, wr_ref[...]) + br_ref[...]


def _tc_mid(acc, denp, bias, g, b, wl, bl, wr, br):
    return pl.pallas_call(
        _tc_mid_body,
        out_shape=(jax.ShapeDtypeStruct((N, D), jnp.float32),) * 2,
    )(acc, denp, bias[None, :], g[None, :], b[None, :], wl, bl[None, :],
      wr, br[None, :])


def _tc_final_body(acc_ref, denp_ref, bias_ref, h_ref, lw_ref, lb_ref,
                   bg_ref, bb_ref, out_ref):
    s = h_ref[...] + _gat_out(acc_ref, denp_ref, bias_ref)
    out = _mm_t(s, lw_ref[...]) + lb_ref[...]
    out_ref[...] = out / jnp.sqrt(1.0 + EPS) * bg_ref[...] + bb_ref[...]


def _tc_final(acc, denp, bias, h, lw, lb, bg, bb):
    return pl.pallas_call(
        _tc_final_body,
        out_shape=jax.ShapeDtypeStruct((N, D), jnp.float32),
    )(acc, denp, bias[None, :], h, lw, lb[None, :], bg[None, :], bb[None, :])


def kernel(x, edge_index, ptr, params):
    del ptr
    loops = jnp.arange(N, dtype=jnp.int32)
    src = jnp.concatenate([edge_index[0].astype(jnp.int32), loops])
    dst = jnp.concatenate([edge_index[1].astype(jnp.int32), loops])
    pad = jnp.zeros((E_PAD - ET,), jnp.int32)
    src = jnp.concatenate([src, pad])
    dst = jnp.concatenate([dst, pad])

    zeros = jnp.zeros((N_ACC, D), jnp.float32)
    h, xl, xr = _tc_init(x, params["proj_W"], params["proj_b"],
                         params["gat"][0]["Wl"], params["gat"][0]["bl"],
                         params["gat"][0]["Wr"], params["gat"][0]["br"])
    out = None
    for i in range(6):
        p = params["gat"][i]
        logits, mxp = _pass_a(src, dst, xl, xr, p["att"])
        mx = _tc_max(mxp)
        acc, denp = _pass_b(src, dst, xl, logits, mx, zeros)
        if i < 5:
            pn = params["gat"][i + 1]
            xl, xr = _tc_mid(acc, denp, p["bias"], params["ln"][i]["g"],
                             params["ln"][i]["b"], pn["Wl"], pn["bl"],
                             pn["Wr"], pn["br"])
        else:
            out = _tc_final(acc, denp, p["bias"], h, params["lin_W"],
                            params["lin_b"], params["bn_g"], params["bn_b"])
    return out
